# baseline (speedup 1.0000x reference)
import jax
import jax.numpy as jnp
from jax import lax
from jax.experimental import pallas as pl
from jax.experimental.pallas import tpu as pltpu
from jax.experimental.pallas import tpu_sc as plsc

B, L = 16384, 200
V, D = 100, 64
ROWS = B * L

NC, NS = 2, 16
NW = NC * NS
NBQ = B // 128
QPW = NBQ // NW
REP = 16


def _table_body(emb_ref, w1_ref, b1_ref, w2_ref, b2_ref, out_ref):
    h = jnp.dot(emb_ref[...], w1_ref[...], preferred_element_type=jnp.float32)
    h = jnp.maximum(h + b1_ref[...], 0.0)
    h = jnp.dot(h, w2_ref[...], preferred_element_type=jnp.float32)
    out_ref[...] = jnp.maximum(h + b2_ref[...], 0.0)


def _mlp_table(emb, w1, b1, w2, b2):
    return pl.pallas_call(
        _table_body,
        out_shape=jax.ShapeDtypeStruct((V, D), jnp.float32),
    )(emb, w1, b1.reshape(1, D), w2, b2.reshape(1, D))


def _expand_body(tabrep_hbm, idxt_hbm, out_hbm,
                 tab_v, idxb, slab0, slab1,
                 isem, wsem0, wsem1):
    wid = lax.axis_index("s") * NC + lax.axis_index("c")
    slab = (slab0, slab1)
    wsem = (wsem0, wsem1)
    iota = lax.iota(jnp.int32, 16)

    pltpu.sync_copy(tabrep_hbm, tab_v)

    def idx_src(l):
        return idxt_hbm.at[pl.ds(l * B + wid * (QPW * 128), QPW * 128)]

    def idx_dst(l):
        return idxb.at[pl.ds((l % 2) * (QPW * 128), QPW * 128)]

    def wait_write(s):
        pltpu.make_async_copy(slab[s], out_hbm.at[0, :, 0, :], wsem[s]).wait()

    def quarter(l, q, hl):
        s = q & 1

        @pl.loop(0, 8, unroll=2)
        def _(g):
            vidx = idxb[pl.ds(hl + q * 128 + g * 16, 16)] * REP + iota
            for db in range(0, D, 16):
                vals = [plsc.load_gather(
                    tab_v.at[pl.ds((db + j) * V * REP, V * REP)], [vidx])
                    for j in range(16)]
                for j in range(16):
                    slab[s][(db + j) // 8, pl.ds((j % 8) * 128 + g * 16, 16)] = vals[j]

        pltpu.async_copy(slab[s], out_hbm.at[l, :, wid * QPW + q, :], wsem[s])

    pltpu.async_copy(idx_src(0), idx_dst(0), isem)

    @pl.loop(0, L)
    def _(l):
        hl = (l % 2) * (QPW * 128)
        pltpu.make_async_copy(idx_src(l), idx_dst(l), isem).wait()

        @pl.when(l < L - 1)
        def _():
            pltpu.async_copy(idx_src(l + 1), idx_dst(l + 1), isem)

        for q in range(QPW):
            if q < 2:
                @pl.when(l > 0)
                def _():
                    wait_write(q & 1)
            else:
                wait_write(q & 1)
            quarter(l, q, hl)

    wait_write(0)
    wait_write(1)


def _sc_expand(tabrep, idxt):
    mesh = plsc.VectorSubcoreMesh(core_axis_name="c", subcore_axis_name="s")
    return pl.kernel(
        _expand_body,
        mesh=mesh,
        compiler_params=pltpu.CompilerParams(
            use_tc_tiling_on_sc=False, needs_layout_passes=False,
            disable_bounds_checks=True),
        out_type=jax.ShapeDtypeStruct((L, D // 8, NBQ, 8 * 128), jnp.float32),
        scratch_types=[
            pltpu.VMEM((V * D * REP,), jnp.float32),
            pltpu.VMEM((2 * QPW * 128,), jnp.int32),
            pltpu.VMEM((D // 8, 8 * 128), jnp.float32),
            pltpu.VMEM((D // 8, 8 * 128), jnp.float32),
            pltpu.SemaphoreType.DMA,
            pltpu.SemaphoreType.DMA,
            pltpu.SemaphoreType.DMA,
        ],
    )(tabrep, idxt)


def kernel(x, emb, W1, b1, W2, b2):
    table = _mlp_table(emb, W1, b1, W2, b2)
    tabrep = jnp.broadcast_to(
        table.T.reshape(V * D, 1), (V * D, REP)).reshape(V * D * REP)
    idxt = x.T.astype(jnp.int32).reshape(ROWS)
    out = _sc_expand(tabrep, idxt)
    return (out.reshape(L, D // 8, NBQ, 8, 128)
            .transpose(2, 4, 0, 1, 3).reshape(B, L, D))


# speedup vs baseline: 1.0274x; 1.0274x over previous
import jax
import jax.numpy as jnp
from jax import lax
from jax.experimental import pallas as pl
from jax.experimental.pallas import tpu as pltpu
from jax.experimental.pallas import tpu_sc as plsc

B, L = 16384, 200
V, D = 100, 64
ROWS = B * L

NC, NS = 2, 16
NW = NC * NS
NBQ = B // 128
QPW = NBQ // NW
REP = 16


def _table_body(emb_ref, w1_ref, b1_ref, w2_ref, b2_ref, out_ref):
    h = jnp.dot(emb_ref[...], w1_ref[...], preferred_element_type=jnp.float32)
    h = jnp.maximum(h + b1_ref[...], 0.0)
    h = jnp.dot(h, w2_ref[...], preferred_element_type=jnp.float32)
    out_ref[...] = jnp.maximum(h + b2_ref[...], 0.0)


def _mlp_table(emb, w1, b1, w2, b2):
    return pl.pallas_call(
        _table_body,
        out_shape=jax.ShapeDtypeStruct((V, D), jnp.float32),
    )(emb, w1, b1.reshape(1, D), w2, b2.reshape(1, D))


def _expand_body(tabrep_hbm, idxt_hbm, out_hbm,
                 tab_v, idxb, slab0, slab1,
                 isem, wsem0, wsem1):
    wid = lax.axis_index("s") * NC + lax.axis_index("c")
    slab = (slab0, slab1)
    wsem = (wsem0, wsem1)
    iota = lax.iota(jnp.int32, 16)

    pltpu.sync_copy(tabrep_hbm, tab_v)

    def idx_src(l):
        return idxt_hbm.at[pl.ds(l * B + wid * (QPW * 128), QPW * 128)]

    def idx_dst(l):
        return idxb.at[pl.ds((l % 2) * (QPW * 128), QPW * 128)]

    def wait_write(s):
        pltpu.make_async_copy(slab[s], out_hbm.at[0, :, 0, :], wsem[s]).wait()

    def quarter(l, q, hl):
        s = q & 1

        @pl.loop(0, 8, unroll=2)
        def _(g):
            vidx = idxb[pl.ds(hl + q * 128 + g * 16, 16)] * REP + iota
            for db in range(0, D, 8):
                vals = [plsc.load_gather(
                    tab_v.at[pl.ds((db + j) * V * REP, V * REP)], [vidx])
                    for j in range(8)]
                for j in range(8):
                    slab[s][db // 8, pl.ds(j * 128 + g * 16, 16)] = vals[j]

        pltpu.async_copy(slab[s], out_hbm.at[l, :, wid * QPW + q, :], wsem[s])

    pltpu.async_copy(idx_src(0), idx_dst(0), isem)

    @pl.loop(0, L)
    def _(l):
        hl = (l % 2) * (QPW * 128)
        pltpu.make_async_copy(idx_src(l), idx_dst(l), isem).wait()

        @pl.when(l < L - 1)
        def _():
            pltpu.async_copy(idx_src(l + 1), idx_dst(l + 1), isem)

        for q in range(QPW):
            if q < 2:
                @pl.when(l > 0)
                def _():
                    wait_write(q & 1)
            else:
                wait_write(q & 1)
            quarter(l, q, hl)

    wait_write(0)
    wait_write(1)


def _sc_expand(tabrep, idxt):
    mesh = plsc.VectorSubcoreMesh(core_axis_name="c", subcore_axis_name="s")
    return pl.kernel(
        _expand_body,
        mesh=mesh,
        compiler_params=pltpu.CompilerParams(
            use_tc_tiling_on_sc=False, needs_layout_passes=False,
            disable_bounds_checks=True),
        out_type=jax.ShapeDtypeStruct((L, D // 8, NBQ, 8 * 128), jnp.float32),
        scratch_types=[
            pltpu.VMEM((V * D * REP,), jnp.float32),
            pltpu.VMEM((2 * QPW * 128,), jnp.int32),
            pltpu.VMEM((D // 8, 8 * 128), jnp.float32),
            pltpu.VMEM((D // 8, 8 * 128), jnp.float32),
            pltpu.SemaphoreType.DMA,
            pltpu.SemaphoreType.DMA,
            pltpu.SemaphoreType.DMA,
        ],
    )(tabrep, idxt)


def kernel(x, emb, W1, b1, W2, b2):
    table = _mlp_table(emb, W1, b1, W2, b2)
    tabrep = jnp.broadcast_to(
        table.T.reshape(V * D, 1), (V * D, REP)).reshape(V * D * REP)
    idxt = x.T.astype(jnp.int32).reshape(ROWS)
    out = _sc_expand(tabrep, idxt)
    return (out.reshape(L, D // 8, NBQ, 8, 128)
            .transpose(2, 4, 0, 1, 3).reshape(B, L, D))
